# baseline (speedup 1.0000x reference)
import functools

import jax
import jax.numpy as jnp
from jax import lax
from jax.experimental import pallas as pl
from jax.experimental.pallas import tpu as pltpu
from jax.experimental.pallas import tpu_sc as plsc

VOCAB = 1000000
D = 32
B = 16384

_info = plsc.get_sparse_core_info()
NC = _info.num_cores
NS = _info.num_subcores
NW = NC * NS
BPW = B // NW
CH = 8
NCH = BPW // CH
OUTW = 256

_mesh = plsc.VectorSubcoreMesh(core_axis_name="c", subcore_axis_name="s")


@functools.partial(
    pl.kernel,
    mesh=_mesh,
    compiler_params=pltpu.CompilerParams(needs_layout_passes=False),
    out_type=jax.ShapeDtypeStruct((2 * D, B), jnp.float32),
    scratch_types=[
        pltpu.VMEM((BPW,), jnp.int32),
        pltpu.VMEM((BPW,), jnp.int32),
        pltpu.VMEM((64,), jnp.float32),
        pltpu.VMEM((CH, D, 128), jnp.float32),
        pltpu.VMEM((CH, D, 128), jnp.float32),
        pltpu.VMEM((CH, D, 128), jnp.float32),
        pltpu.VMEM((D, 128), jnp.float32),
        pltpu.VMEM((2 * D, OUTW), jnp.float32),
        pltpu.SemaphoreType.DMA,
        pltpu.SemaphoreType.DMA,
        pltpu.SemaphoreType.DMA,
    ],
)
def _lookup_concat(uid_hbm, sex_hbm, utab_hbm, stab_hbm, out_hbm,
                   uid_v, sex_v, stab_v, buf0_v, buf1_v, buf2_v, dummy_v,
                   out_v, sem0, sem1, sem2):
    wid = lax.axis_index("s") * NC + lax.axis_index("c")
    base = wid * BPW
    pltpu.sync_copy(uid_hbm.at[wid], uid_v)
    pltpu.sync_copy(sex_hbm.at[wid], sex_v)
    pltpu.sync_copy(stab_hbm, stab_v)

    lanes = lax.iota(jnp.int32, 16)
    bufs = (buf0_v, buf1_v, buf2_v)
    sems = (sem0, sem1, sem2)

    def fire_chunk(q, j0, buf, sem):
        u16 = uid_v[pl.ds((q >> 1) * 16, 16)]
        for j in range(CH):
            u = u16[j0 + j]
            col = pl.multiple_of((u >> 7) * 128, 128)
            for tr in range(D // 8):
                pltpu.async_copy(
                    utab_hbm.at[pl.ds(tr * 8, 8), pl.ds(col, 128)],
                    buf.at[j, pl.ds(tr * 8, 8)], sem)

    def drain(sem):
        for _ in range(CH):
            pltpu.make_async_copy(
                utab_hbm.at[pl.ds(0, D), pl.ds(0, 128)], dummy_v, sem).wait()

    def extract_chunk(q, j0, buf):
        u16 = uid_v[pl.ds((q >> 1) * 16, 16)]
        s16 = sex_v[pl.ds((q >> 1) * 16, 16)]
        for j in range(CH):
            u = u16[j0 + j]
            s = s16[j0 + j]
            lane = u & 127
            col16 = lanes * 0 + ((q * CH + j) & (OUTW - 1))
            for h in range(D // 16):
                x = plsc.load_gather(
                    buf, [lanes * 0 + j, lanes + h * 16, lanes * 0 + lane])
                plsc.store_scatter(out_v, [lanes + h * 16, col16], x)
                y = plsc.load_gather(stab_v, [s * D + h * 16 + lanes])
                plsc.store_scatter(out_v, [lanes + D + h * 16, col16], y)

    fire_chunk(0, 0, bufs[0], sems[0])
    fire_chunk(1, CH, bufs[1], sems[1])

    def body(k, _):
        for r in range(6):
            @pl.when(k % 6 == r)
            def _step(r=r):
                @pl.when(k < NCH - 2)
                def _f():
                    fire_chunk(k + 2, ((r + 2) % 2) * CH,
                               bufs[(r + 2) % 3], sems[(r + 2) % 3])
                drain(sems[r % 3])
                extract_chunk(k, (r % 2) * CH, bufs[r % 3])

        nflushed = (NCH * CH) // OUTW
        for f in range(nflushed):
            @pl.when(k == (f + 1) * (OUTW // CH) - 1)
            def _flush(f=f):
                pltpu.sync_copy(
                    out_v, out_hbm.at[:, pl.ds(base + f * OUTW, OUTW)])
        return 0

    lax.fori_loop(0, NCH, body, 0)


def kernel(user_id, sex, user_table, sex_table):
    uid = user_id.astype(jnp.int32).reshape(NW, BPW)
    sx = sex.astype(jnp.int32).reshape(NW, BPW)
    utab_t = user_table.T
    stab = sex_table.reshape(64)
    out_t = _lookup_concat(uid, sx, utab_t, stab)
    return out_t.T


# speedup vs baseline: 1.0010x; 1.0010x over previous
import functools

import jax
import jax.numpy as jnp
from jax import lax
from jax.experimental import pallas as pl
from jax.experimental.pallas import tpu as pltpu
from jax.experimental.pallas import tpu_sc as plsc

VOCAB = 1000000
D = 32
B = 16384

_info = plsc.get_sparse_core_info()
NC = _info.num_cores
NS = _info.num_subcores
NW = NC * NS
BPW = B // NW
CH = 8
NCH = BPW // CH
OUTW = 256

_mesh = plsc.VectorSubcoreMesh(core_axis_name="c", subcore_axis_name="s")


@functools.partial(
    pl.kernel,
    mesh=_mesh,
    compiler_params=pltpu.CompilerParams(needs_layout_passes=False),
    out_type=jax.ShapeDtypeStruct((2 * D, B), jnp.float32),
    scratch_types=[
        pltpu.VMEM((BPW,), jnp.int32),
        pltpu.VMEM((BPW,), jnp.int32),
        pltpu.VMEM((64,), jnp.float32),
        pltpu.VMEM((CH, D, 128), jnp.float32),
        pltpu.VMEM((CH, D, 128), jnp.float32),
        pltpu.VMEM((CH, D, 128), jnp.float32),
        pltpu.VMEM((D, 128), jnp.float32),
        pltpu.VMEM((2 * D, OUTW), jnp.float32),
        pltpu.SemaphoreType.DMA,
        pltpu.SemaphoreType.DMA,
        pltpu.SemaphoreType.DMA,
    ],
)
def _lookup_concat(uid_hbm, sex_hbm, utab_hbm, stab_hbm, out_hbm,
                   uid_v, sex_v, stab_v, buf0_v, buf1_v, buf2_v, dummy_v,
                   out_v, sem0, sem1, sem2):
    wid = lax.axis_index("s") * NC + lax.axis_index("c")
    base = wid * BPW
    pltpu.sync_copy(uid_hbm.at[wid], uid_v)
    pltpu.sync_copy(sex_hbm.at[wid], sex_v)
    pltpu.sync_copy(stab_hbm, stab_v)

    lanes = lax.iota(jnp.int32, 16)
    bufs = (buf0_v, buf1_v, buf2_v)
    sems = (sem0, sem1, sem2)

    def fire_chunk(q, j0, buf, sem):
        u16 = uid_v[pl.ds((q >> 1) * 16, 16)]
        for j in range(CH):
            u = u16[j0 + j]
            col = pl.multiple_of((u >> 7) * 128, 128)
            pltpu.async_copy(
                utab_hbm.at[pl.ds(0, D), pl.ds(col, 128)], buf.at[j], sem)

    def drain(sem):
        for _ in range(CH):
            pltpu.make_async_copy(
                utab_hbm.at[pl.ds(0, D), pl.ds(0, 128)], dummy_v, sem).wait()

    def extract_chunk(q, j0, buf):
        u16 = uid_v[pl.ds((q >> 1) * 16, 16)]
        s16 = sex_v[pl.ds((q >> 1) * 16, 16)]
        for j in range(CH):
            u = u16[j0 + j]
            s = s16[j0 + j]
            lane = u & 127
            col16 = lanes * 0 + ((q * CH + j) & (OUTW - 1))
            for h in range(D // 16):
                x = plsc.load_gather(
                    buf, [lanes * 0 + j, lanes + h * 16, lanes * 0 + lane])
                plsc.store_scatter(out_v, [lanes + h * 16, col16], x)
                y = plsc.load_gather(stab_v, [s * D + h * 16 + lanes])
                plsc.store_scatter(out_v, [lanes + D + h * 16, col16], y)

    fire_chunk(0, 0, bufs[0], sems[0])
    fire_chunk(1, CH, bufs[1], sems[1])

    def body(k, _):
        for r in range(6):
            @pl.when(k % 6 == r)
            def _step(r=r):
                @pl.when(k < NCH - 2)
                def _f():
                    fire_chunk(k + 2, ((r + 2) % 2) * CH,
                               bufs[(r + 2) % 3], sems[(r + 2) % 3])
                drain(sems[r % 3])
                extract_chunk(k, (r % 2) * CH, bufs[r % 3])

        nflushed = (NCH * CH) // OUTW
        for f in range(nflushed):
            @pl.when(k == (f + 1) * (OUTW // CH) - 1)
            def _flush(f=f):
                pltpu.sync_copy(
                    out_v, out_hbm.at[:, pl.ds(base + f * OUTW, OUTW)])
        return 0

    lax.fori_loop(0, NCH, body, 0)


def kernel(user_id, sex, user_table, sex_table):
    uid = user_id.astype(jnp.int32).reshape(NW, BPW)
    sx = sex.astype(jnp.int32).reshape(NW, BPW)
    utab_t = user_table.T
    stab = sex_table.reshape(64)
    out_t = _lookup_concat(uid, sx, utab_t, stab)
    return out_t.T


# speedup vs baseline: 1.0166x; 1.0156x over previous
import functools

import jax
import jax.numpy as jnp
from jax import lax
from jax.experimental import pallas as pl
from jax.experimental.pallas import tpu as pltpu
from jax.experimental.pallas import tpu_sc as plsc

VOCAB = 1000000
D = 32
B = 16384

_info = plsc.get_sparse_core_info()
NC = _info.num_cores
NS = _info.num_subcores
NW = NC * NS
BPW = B // NW
CH = 8
NCH = BPW // CH
OUTW = 256

_mesh = plsc.VectorSubcoreMesh(core_axis_name="c", subcore_axis_name="s")


@functools.partial(
    pl.kernel,
    mesh=_mesh,
    compiler_params=pltpu.CompilerParams(needs_layout_passes=False),
    out_type=jax.ShapeDtypeStruct((2 * D, B), jnp.float32),
    scratch_types=[
        pltpu.VMEM((BPW,), jnp.int32),
        pltpu.VMEM((BPW,), jnp.int32),
        pltpu.VMEM((64,), jnp.float32),
        pltpu.VMEM((CH, D, 128), jnp.float32),
        pltpu.VMEM((CH, D, 128), jnp.float32),
        pltpu.VMEM((CH, D, 128), jnp.float32),
        pltpu.VMEM((D, 128), jnp.float32),
        pltpu.VMEM((2 * D, OUTW), jnp.float32),
        pltpu.SemaphoreType.DMA,
        pltpu.SemaphoreType.DMA,
        pltpu.SemaphoreType.DMA,
    ],
)
def _lookup_concat(uid_hbm, sex_hbm, utab_hbm, stab_hbm, out_hbm,
                   uid_v, sex_v, stab_v, buf0_v, buf1_v, buf2_v, dummy_v,
                   out_v, sem0, sem1, sem2):
    wid = lax.axis_index("s") * NC + lax.axis_index("c")
    base = wid * BPW
    c0 = pltpu.async_copy(uid_hbm.at[wid], uid_v, sem0)
    c1 = pltpu.async_copy(sex_hbm.at[wid], sex_v, sem0)
    c2 = pltpu.async_copy(stab_hbm, stab_v, sem0)
    c0.wait()
    c1.wait()
    c2.wait()

    lanes = lax.iota(jnp.int32, 16)
    bufs = (buf0_v, buf1_v, buf2_v)
    sems = (sem0, sem1, sem2)

    def fire_chunk(q, j0, buf, sem):
        u16 = uid_v[pl.ds((q >> 1) * 16, 16)]
        for j in range(CH):
            u = u16[j0 + j]
            col = pl.multiple_of((u >> 7) * 128, 128)
            pltpu.async_copy(
                utab_hbm.at[pl.ds(0, D), pl.ds(col, 128)], buf.at[j], sem)

    def drain(sem):
        for _ in range(CH):
            pltpu.make_async_copy(
                utab_hbm.at[pl.ds(0, D), pl.ds(0, 128)], dummy_v, sem).wait()

    def extract_chunk(q, j0, buf):
        u16 = uid_v[pl.ds((q >> 1) * 16, 16)]
        s16 = sex_v[pl.ds((q >> 1) * 16, 16)]
        for j in range(CH):
            u = u16[j0 + j]
            s = s16[j0 + j]
            lane = u & 127
            col16 = lanes * 0 + ((q * CH + j) & (OUTW - 1))
            for h in range(D // 16):
                x = plsc.load_gather(
                    buf, [lanes * 0 + j, lanes + h * 16, lanes * 0 + lane])
                plsc.store_scatter(out_v, [lanes + h * 16, col16], x)
                y = plsc.load_gather(stab_v, [s * D + h * 16 + lanes])
                plsc.store_scatter(out_v, [lanes + D + h * 16, col16], y)

    fire_chunk(0, 0, bufs[0], sems[0])
    fire_chunk(1, CH, bufs[1], sems[1])

    def body(k, _):
        for r in range(6):
            @pl.when(k % 6 == r)
            def _step(r=r):
                @pl.when(k < NCH - 2)
                def _f():
                    fire_chunk(k + 2, ((r + 2) % 2) * CH,
                               bufs[(r + 2) % 3], sems[(r + 2) % 3])
                drain(sems[r % 3])
                extract_chunk(k, (r % 2) * CH, bufs[r % 3])

        nflushed = (NCH * CH) // OUTW
        for f in range(nflushed):
            @pl.when(k == (f + 1) * (OUTW // CH) - 1)
            def _flush(f=f):
                pltpu.sync_copy(
                    out_v, out_hbm.at[:, pl.ds(base + f * OUTW, OUTW)])
        return 0

    lax.fori_loop(0, NCH, body, 0)


def kernel(user_id, sex, user_table, sex_table):
    uid = user_id.astype(jnp.int32).reshape(NW, BPW)
    sx = sex.astype(jnp.int32).reshape(NW, BPW)
    utab_t = user_table.T
    stab = sex_table.reshape(64)
    out_t = _lookup_concat(uid, sx, utab_t, stab)
    return out_t.T
